# Initial kernel scaffold; baseline (speedup 1.0000x reference)
import jax
import jax.numpy as jnp
from jax.experimental import pallas as pl


def kernel(x_source, x_target, neighborhood_indices, neighborhood_values, w_s, w_t):
    raise NotImplementedError("write your pallas kernel here")


# speedup vs baseline: 4.1279x; 4.1279x over previous
import functools

import jax
import jax.numpy as jnp
from jax import lax
from jax.experimental import pallas as pl
from jax.experimental.pallas import tpu as pltpu
from jax.experimental.pallas import tpu_sc as plsc

N_NODES = 10000
D = 128
E = 320000

N_SUBCORES = 16
EDGES_PER_TILE = E // N_SUBCORES
CHUNK = 80
NCHUNK = EDGES_PER_TILE // CHUNK
ROWS_PER_TILE = 624
FCH = 208
NFLUSH = ROWS_PER_TILE // FCH
TAIL_BASE = N_SUBCORES * ROWS_PER_TILE
TAIL_ROWS = N_NODES - TAIL_BASE

_MM_BLK = 1000


def _mm_body(xs_ref, xt_ref, ws_ref, wt_ref, s_out, t_out):
    s_out[...] = jnp.dot(xs_ref[...], ws_ref[...],
                         preferred_element_type=jnp.float32)
    t_out[...] = jnp.dot(xt_ref[...], wt_ref[...],
                         preferred_element_type=jnp.float32)


def _project(x_source, x_target, w_s, w_t):
    return pl.pallas_call(
        _mm_body,
        grid=(N_NODES // _MM_BLK,),
        in_specs=[
            pl.BlockSpec((_MM_BLK, D), lambda i: (i, 0)),
            pl.BlockSpec((_MM_BLK, D), lambda i: (i, 0)),
            pl.BlockSpec((D, D), lambda i: (0, 0)),
            pl.BlockSpec((D, D), lambda i: (0, 0)),
        ],
        out_specs=[
            pl.BlockSpec((_MM_BLK, D), lambda i: (i, 0)),
            pl.BlockSpec((_MM_BLK, D), lambda i: (i, 0)),
        ],
        out_shape=[jax.ShapeDtypeStruct((N_NODES, D), jnp.float32)] * 2,
    )(x_source, x_target, w_s, w_t)


@functools.partial(
    pl.kernel,
    out_type=[jax.ShapeDtypeStruct((N_NODES, D), jnp.float32)] * 2,
    mesh=plsc.VectorSubcoreMesh(core_axis_name="c", subcore_axis_name="s"),
    scratch_types=[
        pltpu.VMEM((CHUNK,), jnp.int32),
        pltpu.VMEM((CHUNK,), jnp.int32),
        pltpu.VMEM((CHUNK,), jnp.float32),
        pltpu.VMEM((CHUNK, D), jnp.float32),
        pltpu.VMEM((FCH, D), jnp.float32),
        pltpu.VMEM_SHARED((N_NODES, D), jnp.float32),
    ],
)
def _sc_spmm(smsg_h, tmsg_h, rows_h, cols_h, vals_h, out_src, out_tgt,
             gidx_v, sidx_v, vals_v, rowbuf, fbuf, acc):
    c = lax.axis_index("c")
    s = lax.axis_index("s")
    rbase = s * ROWS_PER_TILE

    zero16 = jnp.zeros((16,), jnp.float32)

    def _zrow(i, carry):
        for j in range(D // 16):
            fbuf[i, pl.ds(j * 16, 16)] = zero16
        return carry

    lax.fori_loop(0, FCH, _zrow, 0)
    for k in range(NFLUSH):
        pltpu.sync_copy(fbuf, acc.at[pl.ds(rbase + k * FCH, FCH)])

    @pl.when(s == 0)
    def _():
        pltpu.sync_copy(fbuf.at[pl.ds(0, TAIL_ROWS)],
                        acc.at[pl.ds(TAIL_BASE, TAIL_ROWS)])

    plsc.subcore_barrier()

    def _run_direction(table_h, gather_h, scatter_h):
        ebase = s * EDGES_PER_TILE

        def _chunk(k, carry):
            off = ebase + k * CHUNK
            pltpu.sync_copy(gather_h.at[pl.ds(off, CHUNK)], gidx_v)
            pltpu.sync_copy(scatter_h.at[pl.ds(off, CHUNK)], sidx_v)
            pltpu.sync_copy(vals_h.at[pl.ds(off, CHUNK)], vals_v)
            pltpu.sync_copy(table_h.at[gidx_v], rowbuf)

            def _scale(g, cc):
                v16 = vals_v[pl.ds(g * 16, 16)]
                for l in range(16):
                    e = g * 16 + l
                    vb = jnp.full((16,), v16[l], jnp.float32)
                    for j in range(D // 16):
                        sl = pl.ds(j * 16, 16)
                        rowbuf[e, sl] = rowbuf[e, sl] * vb
                return cc

            lax.fori_loop(0, CHUNK // 16, _scale, 0)
            pltpu.sync_copy(rowbuf, acc.at[sidx_v], add=True)
            return carry

        lax.fori_loop(0, NCHUNK, _chunk, 0)

    @pl.when(c == 0)
    def _():
        _run_direction(smsg_h, cols_h, rows_h)

    @pl.when(c == 1)
    def _():
        _run_direction(tmsg_h, rows_h, cols_h)

    plsc.subcore_barrier()

    def _flush(out_h):
        for k in range(NFLUSH):
            r0 = rbase + k * FCH
            pltpu.sync_copy(acc.at[pl.ds(r0, FCH)], fbuf)
            pltpu.sync_copy(fbuf, out_h.at[pl.ds(r0, FCH)])

        @pl.when(s == 0)
        def _():
            pltpu.sync_copy(acc.at[pl.ds(TAIL_BASE, TAIL_ROWS)],
                            fbuf.at[pl.ds(0, TAIL_ROWS)])
            pltpu.sync_copy(fbuf.at[pl.ds(0, TAIL_ROWS)],
                            out_h.at[pl.ds(TAIL_BASE, TAIL_ROWS)])

    @pl.when(c == 0)
    def _():
        _flush(out_tgt)

    @pl.when(c == 1)
    def _():
        _flush(out_src)


def kernel(x_source, x_target, neighborhood_indices, neighborhood_values,
           w_s, w_t):
    s_msg, t_msg = _project(x_source, x_target, w_s, w_t)
    rows = neighborhood_indices[0].astype(jnp.int32)
    cols = neighborhood_indices[1].astype(jnp.int32)
    vals = neighborhood_values.astype(jnp.float32)
    out_src, out_tgt = _sc_spmm(s_msg, t_msg, rows, cols, vals)
    return (out_src, out_tgt)


# speedup vs baseline: 12.7703x; 3.0937x over previous
import functools

import jax
import jax.numpy as jnp
from jax import lax
from jax.experimental import pallas as pl
from jax.experimental.pallas import tpu as pltpu
from jax.experimental.pallas import tpu_sc as plsc

N_NODES = 10000
D = 128
E = 320000

N_SUBCORES = 16
EDGES_PER_TILE = E // N_SUBCORES
CHUNK = 32
NVEC = CHUNK // 16
NBLK = 5
BCHUNK = EDGES_PER_TILE // (NBLK * CHUNK)
NBUF = 5
PF = 3
NGROUP = BCHUNK // NBUF

ROWS_PER_TILE = 624
FCH = 48
NFLUSH = ROWS_PER_TILE // FCH
TAIL_BASE = N_SUBCORES * ROWS_PER_TILE
TAIL_ROWS = N_NODES - TAIL_BASE

_MM_BLK = 1000


def _mm_body(xs_ref, xt_ref, ws_ref, wt_ref, out_ref):
    d = pl.program_id(0)
    x = jnp.where(d == 0, xs_ref[...], xt_ref[...])
    w = jnp.where(d == 0, ws_ref[...], wt_ref[...])
    out_ref[...] = jnp.dot(x, w, preferred_element_type=jnp.float32)


def _project(x_source, x_target, w_s, w_t):
    nb = N_NODES // _MM_BLK
    return pl.pallas_call(
        _mm_body,
        grid=(2, nb),
        in_specs=[
            pl.BlockSpec((_MM_BLK, D), lambda d, i: (i, 0)),
            pl.BlockSpec((_MM_BLK, D), lambda d, i: (i, 0)),
            pl.BlockSpec((D, D), lambda d, i: (0, 0)),
            pl.BlockSpec((D, D), lambda d, i: (0, 0)),
        ],
        out_specs=pl.BlockSpec((_MM_BLK, D), lambda d, i: (d * nb + i, 0)),
        out_shape=jax.ShapeDtypeStruct((2 * N_NODES, D), jnp.float32),
    )(x_source, x_target, w_s, w_t)


@functools.partial(
    pl.kernel,
    out_type=[jax.ShapeDtypeStruct((N_NODES, D), jnp.float32)] * 2,
    mesh=plsc.VectorSubcoreMesh(core_axis_name="c", subcore_axis_name="s"),
    scratch_types=(
        [
            pltpu.VMEM((BCHUNK * CHUNK,), jnp.int32),
            pltpu.VMEM((BCHUNK * CHUNK,), jnp.int32),
            pltpu.VMEM((BCHUNK * CHUNK,), jnp.float32),
        ]
        + [pltpu.VMEM((CHUNK, D), jnp.float32)] * NBUF
        + [
            pltpu.VMEM((FCH, D), jnp.float32),
            pltpu.VMEM_SHARED((N_NODES, D), jnp.float32),
        ]
        + [pltpu.SemaphoreType.DMA] * (2 * NBUF)
    ),
)
def _sc_spmm(table_h, nidx_h, vals_h, out_tgt, out_src,
             gidx_v, sidx_v, vals_v, *rest):
    rbufs = rest[:NBUF]
    fbuf = rest[NBUF]
    acc = rest[NBUF + 1]
    sgs = rest[NBUF + 2:2 * NBUF + 2]
    sss = rest[2 * NBUF + 2:]
    c = lax.axis_index("c")
    s = lax.axis_index("s")
    rbase = s * ROWS_PER_TILE
    zero16 = jnp.zeros((16,), jnp.float32)

    def _zrow(i, carry):
        for j in range(D // 16):
            fbuf[i, pl.ds(j * 16, 16)] = zero16
        return carry

    lax.fori_loop(0, FCH, _zrow, 0)
    for k in range(NFLUSH):
        pltpu.sync_copy(fbuf, acc.at[pl.ds(rbase + k * FCH, FCH)])

    @pl.when(s == 0)
    def _():
        pltpu.sync_copy(fbuf.at[pl.ds(0, TAIL_ROWS)],
                        acc.at[pl.ds(TAIL_BASE, TAIL_ROWS)])

    plsc.subcore_barrier()

    coff = c * N_NODES

    def _gather(k, b):
        for h in range(NVEC):
            greg = gidx_v[pl.ds(k * CHUNK + h * 16, 16)] + coff
            pltpu.async_copy(table_h.at[greg],
                             rbufs[b].at[pl.ds(h * 16, 16)], sgs[b])

    def _wait_gather(k, b):
        for h in range(NVEC):
            greg = gidx_v[pl.ds(k * CHUNK + h * 16, 16)] + coff
            pltpu.make_async_copy(table_h.at[greg],
                                  rbufs[b].at[pl.ds(h * 16, 16)],
                                  sgs[b]).wait()

    def _scatter(k, b):
        for h in range(NVEC):
            sreg = sidx_v[pl.ds(k * CHUNK + h * 16, 16)]
            pltpu.async_copy(rbufs[b].at[pl.ds(h * 16, 16)], acc.at[sreg],
                             sss[b], add=True)

    def _wait_scatter(k, b):
        for h in range(NVEC):
            sreg = sidx_v[pl.ds(k * CHUNK + h * 16, 16)]
            pltpu.make_async_copy(rbufs[b].at[pl.ds(h * 16, 16)],
                                  acc.at[sreg], sss[b]).wait()

    for blk in range(NBLK):
        bs = BCHUNK * CHUNK
        off = s * EDGES_PER_TILE + blk * bs
        pltpu.sync_copy(nidx_h.at[pl.ds((1 - c) * E + off, bs)], gidx_v)
        pltpu.sync_copy(nidx_h.at[pl.ds(c * E + off, bs)], sidx_v)
        pltpu.sync_copy(vals_h.at[pl.ds(off, bs)], vals_v)

        for k in range(PF):
            _gather(k, k)

        def _group(g, carry):
            for b in range(NBUF):
                k = g * NBUF + b
                _wait_gather(k, b)

                bn = (b + PF) % NBUF

                @pl.when(k + PF < BCHUNK)
                def _():
                    @pl.when(k >= NBUF - PF)
                    def _():
                        _wait_scatter(k - (NBUF - PF), bn)

                    _gather(k + PF, bn)

                def _scale(gg, cc):
                    v16 = vals_v[pl.ds(k * CHUNK + gg * 16, 16)]
                    for l in range(16):
                        vb = jnp.full((16,), v16[l], jnp.float32)
                        for j in range(D // 16):
                            sl = pl.ds(j * 16, 16)
                            e = gg * 16 + l
                            rbufs[b][e, sl] = rbufs[b][e, sl] * vb
                    return cc

                lax.fori_loop(0, NVEC, _scale, 0)

                _scatter(k, b)

            return carry

        lax.fori_loop(0, NGROUP, _group, 0)

        for b in range(NBUF):
            _wait_scatter(BCHUNK - NBUF + b, b)

    plsc.subcore_barrier()

    def _flush(out_h):
        for k in range(NFLUSH):
            r0 = rbase + k * FCH
            pltpu.sync_copy(acc.at[pl.ds(r0, FCH)], fbuf)
            pltpu.sync_copy(fbuf, out_h.at[pl.ds(r0, FCH)])

        @pl.when(s == 0)
        def _():
            pltpu.sync_copy(acc.at[pl.ds(TAIL_BASE, TAIL_ROWS)],
                            fbuf.at[pl.ds(0, TAIL_ROWS)])
            pltpu.sync_copy(fbuf.at[pl.ds(0, TAIL_ROWS)],
                            out_h.at[pl.ds(TAIL_BASE, TAIL_ROWS)])

    @pl.when(c == 0)
    def _():
        _flush(out_tgt)

    @pl.when(c == 1)
    def _():
        _flush(out_src)


def kernel(x_source, x_target, neighborhood_indices, neighborhood_values,
           w_s, w_t):
    table = _project(x_source, x_target, w_s, w_t)
    nidx = neighborhood_indices.astype(jnp.int32).reshape(-1)
    vals = neighborhood_values.astype(jnp.float32)
    out_tgt, out_src = _sc_spmm(table, nidx, vals)
    return (out_src, out_tgt)
